# Initial kernel scaffold; baseline (speedup 1.0000x reference)
import jax
import jax.numpy as jnp
from jax.experimental import pallas as pl


def kernel(x, edge_index, batch, W, att_src, att_dst, bias, alpha_train, W1, b1, W2, b2):
    raise NotImplementedError("write your pallas kernel here")


# speedup vs baseline: 17.8670x; 17.8670x over previous
import functools

import jax
import jax.numpy as jnp
from jax import lax
from jax.experimental import pallas as pl
from jax.experimental.pallas import tpu as pltpu
from jax.experimental.pallas import tpu_sc as plsc

N_NODES = 10000
D = 128
H_MID = 64
D_AUG = 144
N_PAD = 10112
NEG = 0.2
DT = 0.1
ITERS = 2
NC = 2
NS = 16
CHUNK = 128
E_TILE = 10368
E_PAD = NC * NS * E_TILE
ROWS_PER_SUB = N_PAD // NS


def _stage_pre_body(coef, z_ref, f_ref, W_ref, att_ref, y_ref, h_ref, a_ref):
    y = z_ref[...] + coef * f_ref[...]
    y_ref[...] = y
    h = jnp.dot(y, W_ref[...], preferred_element_type=jnp.float32)
    lanes = lax.broadcasted_iota(jnp.int32, (N_NODES, D_AUG - D), 1)
    ones_col = jnp.where(lanes == 0, 1.0, 0.0).astype(jnp.float32)
    h_ref[:N_NODES, :] = jnp.concatenate([h, ones_col], axis=1)
    h_ref[N_NODES:, :] = jnp.zeros((N_PAD - N_NODES, D_AUG), jnp.float32)
    a = jnp.dot(h, att_ref[...], preferred_element_type=jnp.float32)
    a_ref[:N_NODES, :] = a
    a_ref[N_NODES:, :] = jnp.full((N_PAD - N_NODES, 2), -1e30, jnp.float32)


def _stage_pre(z, f_prev, coef, W, att2):
    return pl.pallas_call(
        functools.partial(_stage_pre_body, coef),
        out_shape=[
            jax.ShapeDtypeStruct((N_NODES, D), jnp.float32),
            jax.ShapeDtypeStruct((N_PAD, D_AUG), jnp.float32),
            jax.ShapeDtypeStruct((N_PAD, 2), jnp.float32),
        ],
    )(z, f_prev, W, att2)


def _stage_post_body(acc_ref, y_ref, bias_ref, alpha_ref, f_ref):
    s = acc_ref[0] + acc_ref[1]
    sel = lax.broadcasted_iota(jnp.int32, (D_AUG, 1), 0)
    sel = jnp.where(sel == D, 1.0, 0.0).astype(jnp.float32)
    den = jnp.dot(s[:N_NODES, :], sel, preferred_element_type=jnp.float32)
    top = s[:N_NODES, :D]
    gat = top / (den + 1e-16) + bias_ref[...]
    f_ref[...] = alpha_ref[...] * (gat - y_ref[...])


def _stage_post(acc, y, bias, alpha):
    return pl.pallas_call(
        _stage_post_body,
        out_shape=jax.ShapeDtypeStruct((N_NODES, D), jnp.float32),
    )(acc, y, bias, alpha)


def _combine_body(z_ref, k1_ref, k2_ref, k3_ref, k4_ref, o_ref):
    o_ref[...] = z_ref[...] + (DT / 6.0) * (
        k1_ref[...] + 2.0 * k2_ref[...] + 2.0 * k3_ref[...] + k4_ref[...])


def _combine(z, k1, k2, k3, k4):
    return pl.pallas_call(
        _combine_body,
        out_shape=jax.ShapeDtypeStruct((N_NODES, D), jnp.float32),
    )(z, k1, k2, k3, k4)


def _readout_body(z_ref, W1_ref, b1_ref, W2_ref, b2_ref, o_ref):
    r = jnp.tanh(z_ref[...])
    h1 = jnp.dot(r, W1_ref[...], preferred_element_type=jnp.float32) + b1_ref[...]
    h1 = jnp.where(h1 >= 0, h1, 0.01 * h1)
    o = jnp.dot(h1, W2_ref[...], preferred_element_type=jnp.float32) + b2_ref[...]
    o_ref[...] = jnp.where(o >= 0, o, 0.01 * o)


def _readout(z, W1, b1, W2, b2):
    return pl.pallas_call(
        _readout_body,
        out_shape=jax.ShapeDtypeStruct((N_NODES, D), jnp.float32),
    )(z, W1, b1, W2, b2)


def _sc_edge_body(h_hbm, as_hbm, ad_hbm, src_hbm, dst_hbm, zeros_hbm, out_hbm,
                  as_v, ad_v, src_v, dst_v, w_v, rows_v, acc_sh, sem):
    c = lax.axis_index("c")
    s = lax.axis_index("s")
    tile = c * NS + s

    pltpu.sync_copy(zeros_hbm, acc_sh.at[pl.ds(s * ROWS_PER_SUB, ROWS_PER_SUB)])
    pltpu.sync_copy(as_hbm, as_v)
    pltpu.sync_copy(ad_hbm, ad_v)
    plsc.subcore_barrier()

    base0 = tile * E_TILE
    zi = jnp.zeros((16,), jnp.int32)

    def chunk_body(i, carry):
        base = pl.multiple_of(base0 + i * CHUNK, 8)
        pltpu.sync_copy(src_hbm.at[pl.ds(base, CHUNK)], src_v)
        pltpu.sync_copy(dst_hbm.at[pl.ds(base, CHUNK)], dst_v)
        for j in range(CHUNK // 16):
            si = src_v[pl.ds(j * 16, 16)]
            di = dst_v[pl.ds(j * 16, 16)]
            e = plsc.load_gather(as_v, [si]) + plsc.load_gather(ad_v, [di])
            e = jnp.where(e >= 0, e, NEG * e)
            w_v[pl.ds(j * 16, 16)] = jnp.exp(e)
        pltpu.async_copy(h_hbm.at[src_v], rows_v, sem).wait()

        def row_body(r, carry2):
            wb = plsc.load_gather(w_v, [zi + r])
            for k in range(D_AUG // 16):
                rows_v[r, pl.ds(k * 16, 16)] = rows_v[r, pl.ds(k * 16, 16)] * wb
            return carry2

        lax.fori_loop(0, CHUNK, row_body, 0)
        pltpu.sync_copy(rows_v, acc_sh.at[dst_v], add=True)
        return carry

    lax.fori_loop(0, E_TILE // CHUNK, chunk_body, 0)
    plsc.subcore_barrier()
    pltpu.sync_copy(acc_sh.at[pl.ds(s * ROWS_PER_SUB, ROWS_PER_SUB)],
                    out_hbm.at[c, pl.ds(s * ROWS_PER_SUB, ROWS_PER_SUB)])


_sc_edge = functools.partial(
    pl.kernel,
    mesh=plsc.VectorSubcoreMesh(core_axis_name="c", subcore_axis_name="s"),
    out_type=jax.ShapeDtypeStruct((NC, N_PAD, D_AUG), jnp.float32),
    compiler_params=pltpu.CompilerParams(needs_layout_passes=False,
                                         use_tc_tiling_on_sc=False),
    scratch_types=[
        pltpu.VMEM((N_PAD,), jnp.float32),
        pltpu.VMEM((N_PAD,), jnp.float32),
        pltpu.VMEM((CHUNK,), jnp.int32),
        pltpu.VMEM((CHUNK,), jnp.int32),
        pltpu.VMEM((CHUNK,), jnp.float32),
        pltpu.VMEM((CHUNK, D_AUG), jnp.float32),
        pltpu.VMEM_SHARED((N_PAD, D_AUG), jnp.float32),
        pltpu.SemaphoreType.DMA,
    ],
)(_sc_edge_body)


def kernel(x, edge_index, batch, W, att_src, att_dst, bias, alpha_train,
           W1, b1, W2, b2):
    loops = jnp.arange(N_NODES, dtype=jnp.int32)
    n_real = edge_index.shape[1] + N_NODES
    pad = E_PAD - n_real
    src = jnp.concatenate([edge_index[0], loops,
                           jnp.full((pad,), N_NODES, jnp.int32)])
    dst = jnp.concatenate([edge_index[1], loops,
                           jnp.full((pad,), N_NODES, jnp.int32)])

    att2 = jnp.stack([att_src, att_dst], axis=1)
    bias2 = bias[None, :]
    alpha = jnp.broadcast_to(jax.nn.sigmoid(alpha_train), (1, D))
    b1r = b1[None, :]
    b2r = b2[None, :]
    zeros_blk = jnp.zeros((ROWS_PER_SUB, D_AUG), jnp.float32)

    def f_eval(z, f_prev, coef):
        y, h_aug, a2 = _stage_pre(z, f_prev, coef, W, att2)
        acc = _sc_edge(h_aug, a2[:, 0], a2[:, 1], src, dst, zeros_blk)
        return _stage_post(acc, y, bias2, alpha)

    z = x
    for _ in range(ITERS):
        k1 = f_eval(z, z, 0.0)
        k2 = f_eval(z, k1, 0.5 * DT)
        k3 = f_eval(z, k2, 0.5 * DT)
        k4 = f_eval(z, k3, DT)
        z = _combine(z, k1, k2, k3, k4)

    return _readout(z, W1, b1r, W2, b2r)


# speedup vs baseline: 18.1767x; 1.0173x over previous
import functools

import jax
import jax.numpy as jnp
from jax import lax
from jax.experimental import pallas as pl
from jax.experimental.pallas import tpu as pltpu
from jax.experimental.pallas import tpu_sc as plsc

N_NODES = 10000
D = 128
H_MID = 64
D_AUG = 144
N_PAD = 10112
NEG = 0.2
DT = 0.1
ITERS = 2
NC = 2
NS = 16
CHUNK = 128
E_TILE = 10368
E_PAD = NC * NS * E_TILE
ROWS_PER_SUB = N_PAD // NS


def _stage_pre_body(coef, z_ref, f_ref, W_ref, att_ref, y_ref, h_ref, a_ref):
    y = z_ref[...] + coef * f_ref[...]
    y_ref[...] = y
    h = jnp.dot(y, W_ref[...], preferred_element_type=jnp.float32)
    lanes = lax.broadcasted_iota(jnp.int32, (N_NODES, D_AUG - D), 1)
    ones_col = jnp.where(lanes == 0, 1.0, 0.0).astype(jnp.float32)
    h_ref[:N_NODES, :] = jnp.concatenate([h, ones_col], axis=1)
    h_ref[N_NODES:, :] = jnp.zeros((N_PAD - N_NODES, D_AUG), jnp.float32)
    a = jnp.dot(h, att_ref[...], preferred_element_type=jnp.float32)
    a_ref[:N_NODES, :] = a
    a_ref[N_NODES:, :] = jnp.full((N_PAD - N_NODES, 2), -1e30, jnp.float32)


def _stage_pre(z, f_prev, coef, W, att2):
    return pl.pallas_call(
        functools.partial(_stage_pre_body, coef),
        out_shape=[
            jax.ShapeDtypeStruct((N_NODES, D), jnp.float32),
            jax.ShapeDtypeStruct((N_PAD, D_AUG), jnp.float32),
            jax.ShapeDtypeStruct((N_PAD, 2), jnp.float32),
        ],
    )(z, f_prev, W, att2)


def _stage_post_body(acc_ref, y_ref, bias_ref, alpha_ref, f_ref):
    s = acc_ref[0] + acc_ref[1]
    sel = lax.broadcasted_iota(jnp.int32, (D_AUG, 1), 0)
    sel = jnp.where(sel == D, 1.0, 0.0).astype(jnp.float32)
    den = jnp.dot(s[:N_NODES, :], sel, preferred_element_type=jnp.float32)
    top = s[:N_NODES, :D]
    gat = top / (den + 1e-16) + bias_ref[...]
    f_ref[...] = alpha_ref[...] * (gat - y_ref[...])


def _stage_post(acc, y, bias, alpha):
    return pl.pallas_call(
        _stage_post_body,
        out_shape=jax.ShapeDtypeStruct((N_NODES, D), jnp.float32),
    )(acc, y, bias, alpha)


def _combine_body(z_ref, k1_ref, k2_ref, k3_ref, k4_ref, o_ref):
    o_ref[...] = z_ref[...] + (DT / 6.0) * (
        k1_ref[...] + 2.0 * k2_ref[...] + 2.0 * k3_ref[...] + k4_ref[...])


def _combine(z, k1, k2, k3, k4):
    return pl.pallas_call(
        _combine_body,
        out_shape=jax.ShapeDtypeStruct((N_NODES, D), jnp.float32),
    )(z, k1, k2, k3, k4)


def _readout_body(z_ref, W1_ref, b1_ref, W2_ref, b2_ref, o_ref):
    r = jnp.tanh(z_ref[...])
    h1 = jnp.dot(r, W1_ref[...], preferred_element_type=jnp.float32) + b1_ref[...]
    h1 = jnp.where(h1 >= 0, h1, 0.01 * h1)
    o = jnp.dot(h1, W2_ref[...], preferred_element_type=jnp.float32) + b2_ref[...]
    o_ref[...] = jnp.where(o >= 0, o, 0.01 * o)


def _readout(z, W1, b1, W2, b2):
    return pl.pallas_call(
        _readout_body,
        out_shape=jax.ShapeDtypeStruct((N_NODES, D), jnp.float32),
    )(z, W1, b1, W2, b2)


def _sc_edge_body(h_hbm, as_hbm, ad_hbm, src_hbm, dst_hbm, zeros_hbm, out_hbm,
                  as_v, ad_v, src_v, dst_v, w_v, rows_v, acc_sh, sem):
    c = lax.axis_index("c")
    s = lax.axis_index("s")
    tile = c * NS + s

    pltpu.sync_copy(zeros_hbm, acc_sh.at[pl.ds(s * ROWS_PER_SUB, ROWS_PER_SUB)])
    pltpu.sync_copy(as_hbm, as_v)
    pltpu.sync_copy(ad_hbm, ad_v)
    plsc.subcore_barrier()

    base0 = tile * E_TILE
    zi = jnp.zeros((16,), jnp.int32)

    def chunk_body(i, carry):
        base = pl.multiple_of(base0 + i * CHUNK, 8)
        pltpu.sync_copy(src_hbm.at[pl.ds(base, CHUNK)], src_v)
        pltpu.sync_copy(dst_hbm.at[pl.ds(base, CHUNK)], dst_v)
        gather = pltpu.async_copy(h_hbm.at[src_v], rows_v, sem)
        for j in range(CHUNK // 16):
            si = src_v[pl.ds(j * 16, 16)]
            di = dst_v[pl.ds(j * 16, 16)]
            e = plsc.load_gather(as_v, [si]) + plsc.load_gather(ad_v, [di])
            e = jnp.where(e >= 0, e, NEG * e)
            w_v[pl.ds(j * 16, 16)] = jnp.exp(e)
        gather.wait()

        def row_body(r, carry2):
            wb = plsc.load_gather(w_v, [zi + r])
            for k in range(D_AUG // 16):
                rows_v[r, pl.ds(k * 16, 16)] = rows_v[r, pl.ds(k * 16, 16)] * wb
            return carry2

        lax.fori_loop(0, CHUNK, row_body, 0)
        pltpu.sync_copy(rows_v, acc_sh.at[dst_v], add=True)
        return carry

    lax.fori_loop(0, E_TILE // CHUNK, chunk_body, 0)
    plsc.subcore_barrier()
    pltpu.sync_copy(acc_sh.at[pl.ds(s * ROWS_PER_SUB, ROWS_PER_SUB)],
                    out_hbm.at[c, pl.ds(s * ROWS_PER_SUB, ROWS_PER_SUB)])


_sc_edge = functools.partial(
    pl.kernel,
    mesh=plsc.VectorSubcoreMesh(core_axis_name="c", subcore_axis_name="s"),
    out_type=jax.ShapeDtypeStruct((NC, N_PAD, D_AUG), jnp.float32),
    compiler_params=pltpu.CompilerParams(needs_layout_passes=False,
                                         use_tc_tiling_on_sc=False),
    scratch_types=[
        pltpu.VMEM((N_PAD,), jnp.float32),
        pltpu.VMEM((N_PAD,), jnp.float32),
        pltpu.VMEM((CHUNK,), jnp.int32),
        pltpu.VMEM((CHUNK,), jnp.int32),
        pltpu.VMEM((CHUNK,), jnp.float32),
        pltpu.VMEM((CHUNK, D_AUG), jnp.float32),
        pltpu.VMEM_SHARED((N_PAD, D_AUG), jnp.float32),
        pltpu.SemaphoreType.DMA,
    ],
)(_sc_edge_body)


def kernel(x, edge_index, batch, W, att_src, att_dst, bias, alpha_train,
           W1, b1, W2, b2):
    loops = jnp.arange(N_NODES, dtype=jnp.int32)
    n_real = edge_index.shape[1] + N_NODES
    pad = E_PAD - n_real
    src = jnp.concatenate([edge_index[0], loops,
                           jnp.full((pad,), N_NODES, jnp.int32)])
    dst = jnp.concatenate([edge_index[1], loops,
                           jnp.full((pad,), N_NODES, jnp.int32)])

    att2 = jnp.stack([att_src, att_dst], axis=1)
    bias2 = bias[None, :]
    alpha = jnp.broadcast_to(jax.nn.sigmoid(alpha_train), (1, D))
    b1r = b1[None, :]
    b2r = b2[None, :]
    zeros_blk = jnp.zeros((ROWS_PER_SUB, D_AUG), jnp.float32)

    def f_eval(z, f_prev, coef):
        y, h_aug, a2 = _stage_pre(z, f_prev, coef, W, att2)
        acc = _sc_edge(h_aug, a2[:, 0], a2[:, 1], src, dst, zeros_blk)
        return _stage_post(acc, y, bias2, alpha)

    z = x
    for _ in range(ITERS):
        k1 = f_eval(z, z, 0.0)
        k2 = f_eval(z, k1, 0.5 * DT)
        k3 = f_eval(z, k2, 0.5 * DT)
        k4 = f_eval(z, k3, DT)
        z = _combine(z, k1, k2, k3, k4)

    return _readout(z, W1, b1r, W2, b2r)
